# Initial kernel scaffold; baseline (speedup 1.0000x reference)
import jax
import jax.numpy as jnp
from jax.experimental import pallas as pl


def kernel(input, emb_table):
    raise NotImplementedError("write your pallas kernel here")


# speedup vs baseline: 1.2539x; 1.2539x over previous
import functools

import jax
import jax.numpy as jnp
from jax import lax
from jax.experimental import pallas as pl
from jax.experimental.pallas import tpu as pltpu
from jax.experimental.pallas import tpu_sc as plsc

EMBED_DIM = 64
GROUP = 128
NBUF = 4
NUM_CORES = 2
NUM_SUBCORES = 16
NUM_WORKERS = NUM_CORES * NUM_SUBCORES
LANES = 16


@functools.lru_cache(maxsize=None)
def _build_sc_gather(B, D):
    rows_per_w = B // NUM_WORKERS
    groups_per_w = rows_per_w // GROUP
    mesh = plsc.VectorSubcoreMesh(
        core_axis_name="c", subcore_axis_name="s",
        num_cores=NUM_CORES, num_subcores=NUM_SUBCORES)

    @functools.partial(
        pl.kernel,
        out_type=jax.ShapeDtypeStruct((B, D), jnp.float32),
        mesh=mesh,
        compiler_params=pltpu.CompilerParams(
            needs_layout_passes=False, use_tc_tiling_on_sc=False),
        scratch_types=(
            [pltpu.VMEM((groups_per_w, GROUP), jnp.int32)]
            + [pltpu.VMEM((GROUP, D), jnp.float32) for _ in range(NBUF)]
            + [pltpu.SemaphoreType.DMA for _ in range(2 * NBUF)]
        ),
    )
    def sc_gather(table_hbm, idx_hbm, out_hbm, idx_v, *bufs_and_sems):
        bufs = bufs_and_sems[:NBUF]
        gsems = bufs_and_sems[NBUF:2 * NBUF]
        osems = bufs_and_sems[2 * NBUF:]

        wid = lax.axis_index("s") * NUM_CORES + lax.axis_index("c")
        row0 = wid * rows_per_w

        pltpu.sync_copy(idx_hbm.at[pl.ds(wid * groups_per_w, groups_per_w)],
                        idx_v)

        ones = jnp.ones((LANES,), jnp.float32)
        row_ids = [lax.iota(jnp.int32, LANES) + LANES * k
                   for k in range(GROUP // LANES)]

        def gather(j, slot):
            return pltpu.make_async_copy(
                table_hbm.at[idx_v.at[j]], bufs[slot], gsems[slot])

        def writeback(j, slot):
            return pltpu.make_async_copy(
                bufs[slot], out_hbm.at[pl.ds(row0 + j * GROUP, GROUP)],
                osems[slot])

        for b in range(min(NBUF, groups_per_w)):
            gather(b, b).start()

        for j in range(groups_per_w):
            slot = j % NBUF
            gather(j, slot).wait()
            for k in range(GROUP // LANES):
                tok = idx_v[j, pl.ds(LANES * k, LANES)]
                plsc.addupdate_scatter(bufs[slot], [row_ids[k], tok], ones)
            writeback(j, slot).start()
            nj = j + NBUF
            if nj < groups_per_w:
                writeback(j, slot).wait()
                gather(nj, slot).start()

        for j in range(max(0, groups_per_w - NBUF), groups_per_w):
            writeback(j, j % NBUF).wait()

    return sc_gather


def kernel(input, emb_table):
    S0, S1 = input.shape
    B = S0 * S1
    idx2d = input.reshape(B // GROUP, GROUP)
    out = _build_sc_gather(B, emb_table.shape[1])(emb_table, idx2d)
    return out.reshape(S0, S1, emb_table.shape[1])


# speedup vs baseline: 1.2577x; 1.0030x over previous
import functools

import jax
import jax.numpy as jnp
from jax import lax
from jax.experimental import pallas as pl
from jax.experimental.pallas import tpu as pltpu
from jax.experimental.pallas import tpu_sc as plsc

EMBED_DIM = 64
GROUP = 128
GPB = 2
NBUF = 6
PRIME = 3
NUM_CORES = 2
NUM_SUBCORES = 16
NUM_WORKERS = NUM_CORES * NUM_SUBCORES
LANES = 16


@functools.lru_cache(maxsize=None)
def _build_sc_gather(B, D):
    rows_per_w = B // NUM_WORKERS
    groups_per_w = rows_per_w // GROUP
    buf_rows = GPB * GROUP
    nbig = rows_per_w // buf_rows
    mesh = plsc.VectorSubcoreMesh(
        core_axis_name="c", subcore_axis_name="s",
        num_cores=NUM_CORES, num_subcores=NUM_SUBCORES)

    @functools.partial(
        pl.kernel,
        out_type=jax.ShapeDtypeStruct((B, D), jnp.float32),
        mesh=mesh,
        compiler_params=pltpu.CompilerParams(
            needs_layout_passes=False, use_tc_tiling_on_sc=False),
        scratch_types=(
            [pltpu.VMEM((groups_per_w, GROUP), jnp.int32)]
            + [pltpu.VMEM((buf_rows, D), jnp.float32) for _ in range(NBUF)]
            + [pltpu.SemaphoreType.DMA for _ in range(2 * NBUF)]
        ),
    )
    def sc_gather(table_hbm, idx_hbm, out_hbm, idx_v, *bufs_and_sems):
        bufs = bufs_and_sems[:NBUF]
        gsems = bufs_and_sems[NBUF:2 * NBUF]
        osems = bufs_and_sems[2 * NBUF:]

        wid = lax.axis_index("s") * NUM_CORES + lax.axis_index("c")
        row0 = wid * rows_per_w

        pltpu.sync_copy(idx_hbm.at[pl.ds(wid * groups_per_w, groups_per_w)],
                        idx_v)

        ones = jnp.ones((LANES,), jnp.float32)
        row_ids = [lax.iota(jnp.int32, LANES) + LANES * k
                   for k in range(buf_rows // LANES)]

        def gathers(i, slot):
            return [pltpu.make_async_copy(
                        table_hbm.at[idx_v.at[GPB * i + q]],
                        bufs[slot].at[pl.ds(GROUP * q, GROUP)],
                        gsems[slot])
                    for q in range(GPB)]

        def writeback(i, slot):
            return pltpu.make_async_copy(
                bufs[slot], out_hbm.at[pl.ds(row0 + i * buf_rows, buf_rows)],
                osems[slot])

        for b in range(min(PRIME, nbig)):
            for g in gathers(b, b % NBUF):
                g.start()

        for i in range(nbig):
            slot = i % NBUF
            for g in gathers(i, slot):
                g.wait()
            for k in range(buf_rows // LANES):
                tok = idx_v[GPB * i + k // (GROUP // LANES),
                            pl.ds(LANES * (k % (GROUP // LANES)), LANES)]
                plsc.addupdate_scatter(bufs[slot], [row_ids[k], tok], ones)
            writeback(i, slot).start()
            ni = i + PRIME
            if ni < nbig:
                wprev = ni - NBUF
                if wprev >= 0:
                    writeback(wprev, ni % NBUF).wait()
                for g in gathers(ni, ni % NBUF):
                    g.start()

        for i in range(max(0, nbig - NBUF), nbig):
            writeback(i, i % NBUF).wait()

    return sc_gather


def kernel(input, emb_table):
    S0, S1 = input.shape
    B = S0 * S1
    idx2d = input.reshape(B // GROUP, GROUP)
    out = _build_sc_gather(B, emb_table.shape[1])(emb_table, idx2d)
    return out.reshape(S0, S1, emb_table.shape[1])


# speedup vs baseline: 4.8008x; 3.8170x over previous
import functools

import jax
import jax.numpy as jnp
from jax import lax
from jax.experimental import pallas as pl
from jax.experimental.pallas import tpu as pltpu
from jax.experimental.pallas import tpu_sc as plsc

EMBED_DIM = 64
GROUP = 128
GPB = 2
NBUF = 6
PRIME = 3
NUM_CORES = 2
NUM_SUBCORES = 16
NUM_WORKERS = NUM_CORES * NUM_SUBCORES
LANES = 16


@functools.lru_cache(maxsize=None)
def _build_sc_gather(B, V, D):
    rows_per_w = B // NUM_WORKERS
    groups_per_w = rows_per_w // GROUP
    buf_rows = GPB * GROUP
    nbig = rows_per_w // buf_rows
    mesh = plsc.VectorSubcoreMesh(
        core_axis_name="c", subcore_axis_name="s",
        num_cores=NUM_CORES, num_subcores=NUM_SUBCORES)

    @functools.partial(
        pl.kernel,
        out_type=jax.ShapeDtypeStruct((B, D), jnp.float32),
        mesh=mesh,
        compiler_params=pltpu.CompilerParams(
            needs_layout_passes=False, use_tc_tiling_on_sc=False),
        scratch_types=(
            [pltpu.VMEM((groups_per_w, GROUP), jnp.int32),
             pltpu.VMEM((V, D), jnp.float32),
             pltpu.VMEM_SHARED((NUM_SUBCORES * V, D), jnp.float32)]
            + [pltpu.VMEM((buf_rows, D), jnp.float32) for _ in range(NBUF)]
            + [pltpu.SemaphoreType.DMA for _ in range(2 * NBUF)]
        ),
    )
    def sc_gather(table_hbm, idx_hbm, out_hbm, idx_v, table_v, table_sh,
                  *bufs_and_sems):
        bufs = bufs_and_sems[:NBUF]
        gsems = bufs_and_sems[NBUF:2 * NBUF]
        osems = bufs_and_sems[2 * NBUF:]

        sid = lax.axis_index("s")
        wid = sid * NUM_CORES + lax.axis_index("c")
        row0 = wid * rows_per_w

        pltpu.sync_copy(idx_hbm.at[pl.ds(wid * groups_per_w, groups_per_w)],
                        idx_v)

        pltpu.sync_copy(table_hbm, table_v)
        diag0 = lax.iota(jnp.int32, LANES)
        diag1 = diag0 + LANES
        ones = jnp.ones((LANES,), jnp.float32)
        plsc.addupdate_scatter(table_v, [diag0, diag0], ones)
        plsc.addupdate_scatter(table_v, [diag1, diag1], ones,
                               mask=diag1 < V)
        pltpu.sync_copy(table_v, table_sh.at[pl.ds(sid * V, V)])

        off = jnp.full((LANES,), V, jnp.int32) * sid
        for j in range(groups_per_w):
            for k in range(GROUP // LANES):
                sl = pl.ds(LANES * k, LANES)
                idx_v[j, sl] = idx_v[j, sl] + off

        plsc.subcore_barrier()

        def gathers(i, slot):
            return [pltpu.make_async_copy(
                        table_sh.at[idx_v.at[GPB * i + q]],
                        bufs[slot].at[pl.ds(GROUP * q, GROUP)],
                        gsems[slot])
                    for q in range(GPB)]

        def writeback(i, slot):
            return pltpu.make_async_copy(
                bufs[slot], out_hbm.at[pl.ds(row0 + i * buf_rows, buf_rows)],
                osems[slot])

        for b in range(min(PRIME, nbig)):
            for g in gathers(b, b % NBUF):
                g.start()

        for i in range(nbig):
            slot = i % NBUF
            for g in gathers(i, slot):
                g.wait()
            writeback(i, slot).start()
            ni = i + PRIME
            if ni < nbig:
                wprev = ni - NBUF
                if wprev >= 0:
                    writeback(wprev, ni % NBUF).wait()
                for g in gathers(ni, ni % NBUF):
                    g.start()

        for i in range(max(0, nbig - NBUF), nbig):
            writeback(i, i % NBUF).wait()

    return sc_gather


def kernel(input, emb_table):
    S0, S1 = input.shape
    B = S0 * S1
    V, D = emb_table.shape
    idx2d = input.reshape(B // GROUP, GROUP)
    out = _build_sc_gather(B, V, D)(emb_table, idx2d)
    return out.reshape(S0, S1, D)


# speedup vs baseline: 6.3903x; 1.3311x over previous
import functools

import jax
import jax.numpy as jnp
from jax import lax
from jax.experimental import pallas as pl
from jax.experimental.pallas import tpu as pltpu
from jax.experimental.pallas import tpu_sc as plsc

GROUP = 128
NUM_CORES = 2
NUM_SUBCORES = 16
NUM_WORKERS = NUM_CORES * NUM_SUBCORES
LANES = 16
VPAD = 32


@functools.lru_cache(maxsize=None)
def _build_sc_kernel(S0, S1, V, D):
    rows_per_w = S0 // NUM_WORKERS
    groups_per_w = rows_per_w * (S1 // GROUP)
    slab = S1 * D
    chunks = S1 // LANES
    cpg = GROUP // LANES
    mesh = plsc.VectorSubcoreMesh(
        core_axis_name="c", subcore_axis_name="s",
        num_cores=NUM_CORES, num_subcores=NUM_SUBCORES)

    @functools.partial(
        pl.kernel,
        out_type=jax.ShapeDtypeStruct((S0, slab), jnp.float32),
        mesh=mesh,
        compiler_params=pltpu.CompilerParams(
            needs_layout_passes=False, use_tc_tiling_on_sc=False),
        scratch_types=(
            pltpu.VMEM((groups_per_w, GROUP), jnp.int32),
            pltpu.VMEM((VPAD, D), jnp.float32),
            pltpu.VMEM((D * VPAD,), jnp.float32),
            pltpu.VMEM((slab,), jnp.float32),
            pltpu.VMEM((slab,), jnp.float32),
            pltpu.SemaphoreType.DMA,
            pltpu.SemaphoreType.DMA,
        ),
    )
    def sc_fill(table_hbm, idx_hbm, out_hbm, idx_v, table_v, tableT, buf0,
                buf1, sem0, sem1):
        sid = lax.axis_index("s")
        wid = sid * NUM_CORES + lax.axis_index("c")
        i0 = wid * rows_per_w

        pltpu.sync_copy(idx_hbm.at[pl.ds(wid * groups_per_w, groups_per_w)],
                        idx_v)

        pltpu.sync_copy(table_hbm, table_v.at[pl.ds(0, V)])
        diag0 = lax.iota(jnp.int32, LANES)
        diag1 = diag0 + LANES
        ones = jnp.ones((LANES,), jnp.float32)
        plsc.addupdate_scatter(table_v, [diag0, diag0], ones)
        plsc.addupdate_scatter(table_v, [diag1, diag1], ones,
                               mask=diag1 < V)

        iota16 = lax.iota(jnp.int32, LANES)
        for d in range(D):
            dvec = jnp.full((LANES,), d, jnp.int32)
            v1 = plsc.load_gather(table_v, [iota16, dvec])
            tableT[pl.ds(VPAD * d, LANES)] = v1
            v2 = plsc.load_gather(table_v, [iota16 + LANES, dvec])
            tableT[pl.ds(VPAD * d + LANES, LANES)] = v2

        bufs = (buf0, buf1)
        sems = (sem0, sem1)

        def fill_row(ii, buf):
            def chunk_body(ch, _):
                jt = ch // cpg
                qq = ch - jt * cpg
                tokv = idx_v[(S1 // GROUP) * ii + jt,
                             pl.dslice(qq * LANES, LANES)]
                tj = jt * (8 * GROUP) + qq * LANES
                for d in range(D):
                    v = plsc.load_gather(tableT.at[pl.ds(VPAD * d, VPAD)],
                                         [tokv])
                    off = (d // 8) * ((S1 // GROUP) * 8 * GROUP) \
                        + (d % 8) * GROUP
                    buf[pl.ds(tj + off, LANES)] = v
                return 0

            lax.fori_loop(0, chunks, chunk_body, 0)

        def wb(ii, buf, sem):
            return pltpu.make_async_copy(buf, out_hbm.at[i0 + ii], sem)

        def row_pair(p, _):
            ii_a = p * 2
            ii_b = ii_a + 1

            @pl.when(p > 0)
            def _():
                wb(ii_a - 2, buf0, sem0).wait()
            fill_row(ii_a, buf0)
            wb(ii_a, buf0, sem0).start()

            @pl.when(p > 0)
            def _():
                wb(ii_b - 2, buf1, sem1).wait()
            fill_row(ii_b, buf1)
            wb(ii_b, buf1, sem1).start()
            return 0

        lax.fori_loop(0, rows_per_w // 2, row_pair, 0)
        wb(rows_per_w - 2, buf0, sem0).wait()
        wb(rows_per_w - 1, buf1, sem1).wait()

    return sc_fill


def kernel(input, emb_table):
    S0, S1 = input.shape
    V, D = emb_table.shape
    idx2d = input.reshape((S0 * S1) // GROUP, GROUP)
    out = _build_sc_kernel(S0, S1, V, D)(emb_table, idx2d)
    out = out.reshape(S0, D // 8, S1 // GROUP, 8, GROUP)
    out = out.transpose(0, 2, 4, 1, 3).reshape(S0, S1, D)
    return out


# speedup vs baseline: 13.6456x; 2.1353x over previous
import functools

import jax
import jax.numpy as jnp
from jax import lax
from jax.experimental import pallas as pl
from jax.experimental.pallas import tpu as pltpu
from jax.experimental.pallas import tpu_sc as plsc

GROUP = 128
NUM_CORES = 2
NUM_SUBCORES = 16
NUM_WORKERS = NUM_CORES * NUM_SUBCORES
LANES = 16
VPAD = 32


@functools.lru_cache(maxsize=None)
def _build_sc_kernel(S0, S1, V, D):
    rows_per_w = S0 // NUM_WORKERS
    groups_per_w = rows_per_w * (S1 // GROUP)
    slab = S1 * D
    chunks = S1 // LANES
    cpg = GROUP // LANES
    mesh = plsc.VectorSubcoreMesh(
        core_axis_name="c", subcore_axis_name="s",
        num_cores=NUM_CORES, num_subcores=NUM_SUBCORES)

    @functools.partial(
        pl.kernel,
        out_type=jax.ShapeDtypeStruct((S0, slab), jnp.float32),
        mesh=mesh,
        compiler_params=pltpu.CompilerParams(
            needs_layout_passes=False, use_tc_tiling_on_sc=False),
        scratch_types=(
            pltpu.VMEM((groups_per_w, GROUP), jnp.int32),
            pltpu.VMEM((VPAD, D), jnp.float32),
            pltpu.VMEM((D * VPAD,), jnp.float32),
            pltpu.VMEM((slab,), jnp.float32),
            pltpu.VMEM((slab,), jnp.float32),
            pltpu.SemaphoreType.DMA,
            pltpu.SemaphoreType.DMA,
        ),
    )
    def sc_fill(table_hbm, idx_hbm, out_hbm, idx_v, table_v, tableT, buf0,
                buf1, sem0, sem1):
        sid = lax.axis_index("s")
        wid = sid * NUM_CORES + lax.axis_index("c")
        i0 = wid * rows_per_w

        pltpu.sync_copy(idx_hbm.at[pl.ds(wid * groups_per_w, groups_per_w)],
                        idx_v)

        pltpu.sync_copy(table_hbm, table_v.at[pl.ds(0, V)])
        diag0 = lax.iota(jnp.int32, LANES)
        diag1 = diag0 + LANES
        ones = jnp.ones((LANES,), jnp.float32)
        plsc.addupdate_scatter(table_v, [diag0, diag0], ones)
        plsc.addupdate_scatter(table_v, [diag1, diag1], ones,
                               mask=diag1 < V)

        iota16 = lax.iota(jnp.int32, LANES)
        for d in range(D):
            dvec = jnp.full((LANES,), d, jnp.int32)
            v1 = plsc.load_gather(table_v, [iota16, dvec])
            tableT[pl.ds(VPAD * d, LANES)] = v1
            v2 = plsc.load_gather(table_v, [iota16 + LANES, dvec])
            tableT[pl.ds(VPAD * d + LANES, LANES)] = v2

        bufs = (buf0, buf1)
        sems = (sem0, sem1)

        def fill_row(ii, buf):
            ilv = 4

            def chunk_body(cg, _):
                toks = []
                tjs = []
                for u in range(ilv):
                    ch = cg * ilv + u
                    jt = ch // cpg
                    qq = ch - jt * cpg
                    toks.append(idx_v[(S1 // GROUP) * ii + jt,
                                      pl.dslice(qq * LANES, LANES)])
                    tjs.append(jt * (8 * GROUP) + qq * LANES)
                for d in range(D):
                    off = (d // 8) * ((S1 // GROUP) * 8 * GROUP) \
                        + (d % 8) * GROUP
                    vs = [plsc.load_gather(
                              tableT.at[pl.ds(VPAD * d, VPAD)], [toks[u]])
                          for u in range(ilv)]
                    for u in range(ilv):
                        buf[pl.ds(tjs[u] + off, LANES)] = vs[u]
                return 0

            lax.fori_loop(0, chunks // ilv, chunk_body, 0)

        def wb(ii, buf, sem):
            return pltpu.make_async_copy(buf, out_hbm.at[i0 + ii], sem)

        def row_pair(p, _):
            ii_a = p * 2
            ii_b = ii_a + 1

            @pl.when(p > 0)
            def _():
                wb(ii_a - 2, buf0, sem0).wait()
            fill_row(ii_a, buf0)
            wb(ii_a, buf0, sem0).start()

            @pl.when(p > 0)
            def _():
                wb(ii_b - 2, buf1, sem1).wait()
            fill_row(ii_b, buf1)
            wb(ii_b, buf1, sem1).start()
            return 0

        lax.fori_loop(0, rows_per_w // 2, row_pair, 0)
        wb(rows_per_w - 2, buf0, sem0).wait()
        wb(rows_per_w - 1, buf1, sem1).wait()

    return sc_fill


def kernel(input, emb_table):
    S0, S1 = input.shape
    V, D = emb_table.shape
    idx2d = input.reshape((S0 * S1) // GROUP, GROUP)
    out = _build_sc_kernel(S0, S1, V, D)(emb_table, idx2d)
    out = out.reshape(S0, D // 8, S1 // GROUP, 8, GROUP)
    out = out.transpose(0, 2, 4, 1, 3).reshape(S0, S1, D)
    return out
